# baseline (speedup 1.0000x reference)
import functools

import jax
import jax.numpy as jnp
from jax import lax
from jax.experimental import pallas as pl
from jax.experimental.pallas import tpu as pltpu
from jax.experimental.pallas import tpu_sc as plsc

_NC = 2
_NS = 16
_NW = _NC * _NS
_LANE = 128
_K = 4
_PAD_CHUNKS = 8


@functools.lru_cache(maxsize=None)
def _make_gather_scatter(n_out_rows, rows_w):
    mesh = plsc.VectorSubcoreMesh(core_axis_name="c", subcore_axis_name="s")

    @functools.partial(
        pl.kernel,
        mesh=mesh,
        compiler_params=pltpu.CompilerParams(use_tc_tiling_on_sc=False),
        out_type=jax.ShapeDtypeStruct((n_out_rows, _LANE), jnp.float32),
        scratch_types=[
            pltpu.VMEM((rows_w, _LANE), jnp.int32),
            pltpu.VMEM((rows_w, _LANE), jnp.int32),
        ]
        + [pltpu.VMEM((_LANE, _LANE), jnp.float32) for _ in range(_K)]
        + [pltpu.SemaphoreType.DMA for _ in range(2 * _K)],
    )
    def gs(w_hbm, idx_hbm, vidx_hbm, out_hbm, idx_v, vidx_v, *rest):
        ring = rest[:_K]
        sem_g = rest[_K:2 * _K]
        sem_s = rest[2 * _K:]

        wid = lax.axis_index("s") * _NC + lax.axis_index("c")
        r0 = wid * rows_w
        pltpu.sync_copy(idx_hbm.at[pl.ds(r0, rows_w)], idx_v)
        pltpu.sync_copy(vidx_hbm.at[pl.ds(r0, rows_w)], vidx_v)

        def issue_gather(chunk, slot):
            pltpu.async_copy(
                w_hbm.at[idx_v.at[chunk]], ring[slot], sem_g[slot]
            )

        for s in range(_K):
            issue_gather(s, s)

        def step(t, carry):
            for s in range(_K):
                chunk = t * _K + s
                pltpu.make_async_copy(
                    w_hbm.at[pl.ds(0, _LANE)], ring[s], sem_g[s]
                ).wait()
                pltpu.async_copy(
                    ring[s], out_hbm.at[vidx_v.at[chunk]], sem_s[s]
                )
                prev = (s - 1) % _K

                @pl.when(chunk >= 1)
                def _():
                    pltpu.make_async_copy(
                        ring[prev], out_hbm.at[pl.ds(0, _LANE)], sem_s[prev]
                    ).wait()

                @pl.when((chunk >= 1) & (chunk + _K - 1 < rows_w))
                def _():
                    issue_gather(chunk + _K - 1, prev)

            return carry

        lax.fori_loop(0, rows_w // _K, step, 0)
        last = (rows_w - 1) % _K
        pltpu.make_async_copy(
            ring[last], out_hbm.at[pl.ds(0, _LANE)], sem_s[last]
        ).wait()

    return gs


def kernel(token_ids, weight):
    b, f = token_ids.shape
    v, d = weight.shape
    n = b * f

    step = -(-v // _PAD_CHUNKS)
    step += (-step) % 8
    wpad = jnp.concatenate(
        [
            jnp.pad(
                weight[c * step:min((c + 1) * step, v)],
                ((0, 0), (0, _LANE - d)),
            )
            for c in range(-(-v // step))
        ],
        axis=0,
    )

    fp = -(-f // 8) * 8
    idx = token_ids.reshape(n).astype(jnp.int32)
    pos = jnp.arange(n, dtype=jnp.int32)
    vidx = (pos // f) * fp + pos % f

    chunk = _NW * _LANE
    n_pad = -(-n // chunk) * chunk
    if n_pad != n:
        pad = n_pad - n
        idx = jnp.concatenate([idx, jnp.zeros((pad,), jnp.int32)])
        vidx = jnp.concatenate(
            [vidx, jnp.full((pad,), b * fp - 1, jnp.int32)]
        )

    idx2 = idx.reshape(n_pad // _LANE, _LANE)
    vidx2 = vidx.reshape(n_pad // _LANE, _LANE)
    rows_w = (n_pad // _LANE) // _NW

    outp = _make_gather_scatter(b * fp, rows_w)(wpad, idx2, vidx2)
    out = outp.reshape(b, fp, _LANE)[:, :f, :d]
    return out


# speedup vs baseline: 1.3924x; 1.3924x over previous
import functools

import jax
import jax.numpy as jnp
from jax import lax
from jax.experimental import pallas as pl
from jax.experimental.pallas import tpu as pltpu
from jax.experimental.pallas import tpu_sc as plsc

_NC = 2
_NS = 16
_NW = _NC * _NS
_LANE = 128
_K = 4
_PAD_CHUNKS = 8


@functools.lru_cache(maxsize=None)
def _make_gather_scatter(n_out_rows, rows_w):
    mesh = plsc.VectorSubcoreMesh(core_axis_name="c", subcore_axis_name="s")

    @functools.partial(
        pl.kernel,
        mesh=mesh,
        compiler_params=pltpu.CompilerParams(use_tc_tiling_on_sc=False),
        out_type=jax.ShapeDtypeStruct((n_out_rows, _LANE), jnp.float32),
        scratch_types=[
            pltpu.VMEM((rows_w, _LANE), jnp.int32),
            pltpu.VMEM((rows_w, _LANE), jnp.int32),
        ]
        + [pltpu.VMEM((_LANE, _LANE), jnp.float32) for _ in range(_K)]
        + [pltpu.SemaphoreType.DMA for _ in range(2 * _K)],
    )
    def gs(w_hbm, idx_hbm, vidx_hbm, out_hbm, idx_v, vidx_v, *rest):
        ring = rest[:_K]
        sem_g = rest[_K:2 * _K]
        sem_s = rest[2 * _K:]

        wid = lax.axis_index("s") * _NC + lax.axis_index("c")
        r0 = wid * rows_w
        pltpu.sync_copy(idx_hbm.at[pl.ds(r0, rows_w)], idx_v)
        pltpu.sync_copy(vidx_hbm.at[pl.ds(r0, rows_w)], vidx_v)

        def issue_gather(chunk, slot):
            pltpu.async_copy(
                w_hbm.at[idx_v.at[chunk]], ring[slot], sem_g[slot]
            )

        for s in range(_K):
            issue_gather(s, s)

        def step(t, carry):
            for s in range(_K):
                chunk = t * _K + s
                pltpu.make_async_copy(
                    w_hbm.at[pl.ds(0, _LANE)], ring[s], sem_g[s]
                ).wait()
                pltpu.async_copy(
                    ring[s], out_hbm.at[vidx_v.at[chunk]], sem_s[s]
                )
                prev = (s - 1) % _K

                @pl.when(chunk >= 1)
                def _():
                    pltpu.make_async_copy(
                        ring[prev], out_hbm.at[pl.ds(0, _LANE)], sem_s[prev]
                    ).wait()

                @pl.when((chunk >= 1) & (chunk + _K - 1 < rows_w))
                def _():
                    issue_gather(chunk + _K - 1, prev)

            return carry

        lax.fori_loop(0, rows_w // _K, step, 0)
        last = (rows_w - 1) % _K
        pltpu.make_async_copy(
            ring[last], out_hbm.at[pl.ds(0, _LANE)], sem_s[last]
        ).wait()

    return gs


def kernel(token_ids, weight):
    b, f = token_ids.shape
    v, d = weight.shape
    n = b * f

    wpad = jnp.concatenate(
        [weight, jnp.zeros((v, _LANE - d), jnp.float32)], axis=1
    )

    fp = -(-f // 8) * 8
    idx = token_ids.reshape(n).astype(jnp.int32)
    pos = jnp.arange(n, dtype=jnp.int32)
    vidx = (pos // f) * fp + pos % f

    chunk = _NW * _LANE
    n_pad = -(-n // chunk) * chunk
    if n_pad != n:
        pad = n_pad - n
        idx = jnp.concatenate([idx, jnp.zeros((pad,), jnp.int32)])
        vidx = jnp.concatenate(
            [vidx, jnp.full((pad,), b * fp - 1, jnp.int32)]
        )

    idx2 = idx.reshape(n_pad // _LANE, _LANE)
    vidx2 = vidx.reshape(n_pad // _LANE, _LANE)
    rows_w = (n_pad // _LANE) // _NW

    outp = _make_gather_scatter(b * fp, rows_w)(wpad, idx2, vidx2)
    out = outp.reshape(b, fp, _LANE)[:, :f, :d]
    return out


# speedup vs baseline: 1.5394x; 1.1055x over previous
import functools

import jax
import jax.numpy as jnp
from jax import lax
from jax.experimental import pallas as pl
from jax.experimental.pallas import tpu as pltpu
from jax.experimental.pallas import tpu_sc as plsc

_NC = 2
_NS = 16
_NW = _NC * _NS
_LANE = 128
_K = 4


@functools.lru_cache(maxsize=None)
def _make_gather_scatter(n_out_rows, rows_w, d):
    mesh = plsc.VectorSubcoreMesh(core_axis_name="c", subcore_axis_name="s")

    @functools.partial(
        pl.kernel,
        mesh=mesh,
        compiler_params=pltpu.CompilerParams(use_tc_tiling_on_sc=False),
        out_type=jax.ShapeDtypeStruct((n_out_rows, d), jnp.float32),
        scratch_types=[
            pltpu.VMEM((rows_w, _LANE), jnp.int32),
            pltpu.VMEM((rows_w, _LANE), jnp.int32),
        ]
        + [pltpu.VMEM((_LANE, d), jnp.float32) for _ in range(_K)]
        + [pltpu.SemaphoreType.DMA for _ in range(2 * _K)],
    )
    def gs(w_hbm, idx_hbm, vidx_hbm, out_hbm, idx_v, vidx_v, *rest):
        ring = rest[:_K]
        sem_g = rest[_K:2 * _K]
        sem_s = rest[2 * _K:]

        wid = lax.axis_index("s") * _NC + lax.axis_index("c")
        r0 = wid * rows_w
        pltpu.sync_copy(idx_hbm.at[pl.ds(r0, rows_w)], idx_v)
        pltpu.sync_copy(vidx_hbm.at[pl.ds(r0, rows_w)], vidx_v)

        def issue_gather(chunk, slot):
            pltpu.async_copy(
                w_hbm.at[idx_v.at[chunk]], ring[slot], sem_g[slot]
            )

        for s in range(_K):
            issue_gather(s, s)

        def step(t, carry):
            for s in range(_K):
                chunk = t * _K + s
                pltpu.make_async_copy(
                    w_hbm.at[pl.ds(0, _LANE)], ring[s], sem_g[s]
                ).wait()
                pltpu.async_copy(
                    ring[s], out_hbm.at[vidx_v.at[chunk]], sem_s[s]
                )
                prev = (s - 1) % _K

                @pl.when(chunk >= 1)
                def _():
                    pltpu.make_async_copy(
                        ring[prev], out_hbm.at[pl.ds(0, _LANE)], sem_s[prev]
                    ).wait()

                @pl.when((chunk >= 1) & (chunk + _K - 1 < rows_w))
                def _():
                    issue_gather(chunk + _K - 1, prev)

            return carry

        lax.fori_loop(0, rows_w // _K, step, 0)
        last = (rows_w - 1) % _K
        pltpu.make_async_copy(
            ring[last], out_hbm.at[pl.ds(0, _LANE)], sem_s[last]
        ).wait()

    return gs


def kernel(token_ids, weight):
    b, f = token_ids.shape
    v, d = weight.shape
    n = b * f

    wpad = jnp.pad(weight, ((0, 0), (0, _LANE - d)))
    w2 = wpad.reshape(v * _LANE // d, d)

    fp = -(-f // 8) * 8
    idx = 2 * token_ids.reshape(n).astype(jnp.int32)
    pos = jnp.arange(n, dtype=jnp.int32)
    vidx = 2 * ((pos // f) * fp + pos % f)

    chunk = _NW * _LANE
    n_pad = -(-n // chunk) * chunk
    if n_pad != n:
        pad = n_pad - n
        idx = jnp.concatenate([idx, jnp.zeros((pad,), jnp.int32)])
        vidx = jnp.concatenate(
            [vidx, jnp.full((pad,), 2 * b * fp - 1, jnp.int32)]
        )

    idx2 = idx.reshape(n_pad // _LANE, _LANE)
    vidx2 = vidx.reshape(n_pad // _LANE, _LANE)
    rows_w = (n_pad // _LANE) // _NW

    outp = _make_gather_scatter(2 * b * fp * _LANE // (2 * d), rows_w, d)(
        w2, idx2, vidx2
    )
    out = outp.reshape(b, fp, _LANE)[:, :f, :d]
    return out
